# baseline (speedup 1.0000x reference)
import functools

import jax
import jax.numpy as jnp
from jax import lax
from jax.experimental import pallas as pl
from jax.experimental.pallas import tpu as pltpu
from jax.experimental.pallas import tpu_sc as plsc

V = 1_000_000
D = 32
NC = 2
NS = 16
NW = NC * NS

CE = 128
NFULL = 7812
TAIL_E = V - NFULL * 128
NSLOT_A = 4

NU = 26 * 128
U_PER_W = NU // NW
NSLOT_B = 8


def _relayout(table_t, tail_flat):
    mesh = plsc.VectorSubcoreMesh(core_axis_name="c", subcore_axis_name="s")

    @functools.partial(
        pl.kernel,
        mesh=mesh,
        out_type=jax.ShapeDtypeStruct((V * D,), jnp.float32),
        compiler_params=pltpu.CompilerParams(needs_layout_passes=False),
        scratch_types=(
            [pltpu.VMEM((8, CE), jnp.float32) for _ in range(4 * NSLOT_A)]
            + [pltpu.VMEM((CE * D,), jnp.float32) for _ in range(NSLOT_A)]
            + [pltpu.SemaphoreType.DMA for _ in range(2 * NSLOT_A)]
        ),
    )
    def k(tab_hbm, tail_hbm, out_hbm, *scratch):
        strips = [scratch[4 * b : 4 * b + 4] for b in range(NSLOT_A)]
        rows = scratch[4 * NSLOT_A : 5 * NSLOT_A]
        gsem = scratch[5 * NSLOT_A : 6 * NSLOT_A]
        ssem = scratch[6 * NSLOT_A : 7 * NSLOT_A]

        wid = lax.axis_index("s") * NC + lax.axis_index("c")
        cstart = 244 * wid + 4 * jnp.minimum(wid, 1)
        nch = 244 + 4 * jnp.where(wid == 0, 1, 0)

        def start_reads(n, b):
            e0 = (cstart + n) * CE
            for tr in range(4):
                pltpu.async_copy(
                    tab_hbm.at[pl.ds(8 * tr, 8), pl.ds(e0, CE)],
                    strips[b][tr], gsem[b])

        def wait_reads(n, b):
            e0 = (cstart + n) * CE
            for tr in range(4):
                pltpu.make_async_copy(
                    tab_hbm.at[pl.ds(8 * tr, 8), pl.ds(e0, CE)],
                    strips[b][tr], gsem[b]).wait()

        def start_write(n, b):
            e0 = (cstart + n) * CE
            pltpu.async_copy(rows[b], out_hbm.at[pl.ds(e0 * D, CE * D)],
                             ssem[b])

        def wait_write(n, b):
            e0 = (cstart + n) * CE
            pltpu.make_async_copy(
                rows[b], out_hbm.at[pl.ds(e0 * D, CE * D)], ssem[b]).wait()

        ibase = lax.iota(jnp.int32, 16) * D

        def transpose_chunk(b):
            @plsc.parallel_loop(0, 8, unroll=2)
            def _(l0):
                base = ibase + 512 * l0
                vals = [strips[b][c // 8][c % 8, pl.ds(l0 * 16, 16)]
                        for c in range(D)]
                for c in range(D):
                    plsc.store_scatter(rows[b], [base + c], vals[c])

        for b in range(NSLOT_A):
            start_reads(b, b)

        @pl.loop(0, nch, step=NSLOT_A)
        def _(ch):
            for b in range(NSLOT_A):
                n = ch + b
                wait_reads(n, b)
                @pl.when(n >= NSLOT_A)
                def _():
                    wait_write(n - NSLOT_A, b)
                transpose_chunk(b)
                start_write(n, b)
                @pl.when(n + NSLOT_A < nch)
                def _():
                    start_reads(n + NSLOT_A, b)

        for b in range(NSLOT_A):
            wait_write(nch - NSLOT_A + b, b)

        @pl.when(wid == 0)
        def _():
            pltpu.sync_copy(tail_hbm, rows[0].at[pl.ds(0, TAIL_E * D)])
            pltpu.sync_copy(rows[0].at[pl.ds(0, TAIL_E * D)],
                            out_hbm.at[pl.ds(NFULL * 128 * D, TAIL_E * D)])

    return k(table_t, tail_flat)


def _gather_t(table_r, idx_t):
    mesh = plsc.VectorSubcoreMesh(core_axis_name="c", subcore_axis_name="s")

    @functools.partial(
        pl.kernel,
        mesh=mesh,
        out_type=jax.ShapeDtypeStruct((26, 4, 128, 1024), jnp.float32),
        compiler_params=pltpu.CompilerParams(
            use_tc_tiling_on_sc=False, needs_layout_passes=False),
        scratch_types=(
            [pltpu.VMEM((U_PER_W, 128), jnp.int32)]
            + [pltpu.VMEM((128, D), jnp.float32) for _ in range(NSLOT_B)]
            + [pltpu.VMEM((4096,), jnp.float32) for _ in range(NSLOT_B)]
            + [pltpu.SemaphoreType.DMA for _ in range(2 * NSLOT_B)]
        ),
    )
    def k(table_hbm, idx_hbm, out_hbm, *scratch):
        idx_v = scratch[0]
        rows = scratch[1 : 1 + NSLOT_B]
        tbuf = scratch[1 + NSLOT_B : 1 + 2 * NSLOT_B]
        gsem = scratch[1 + 2 * NSLOT_B : 1 + 3 * NSLOT_B]
        ssem = scratch[1 + 3 * NSLOT_B : 1 + 4 * NSLOT_B]

        wid = lax.axis_index("s") * NC + lax.axis_index("c")
        ubase = wid * U_PER_W
        pltpu.sync_copy(idx_hbm.at[pl.ds(ubase, U_PER_W)], idx_v)

        def unit_ct(n):
            u = ubase + n
            return lax.div(u, 128), lax.rem(u, 128)

        def start_gather(n, b):
            pltpu.async_copy(table_hbm.at[idx_v.at[n]], rows[b], gsem[b])

        def wait_gather(n, b):
            pltpu.make_async_copy(
                table_hbm.at[idx_v.at[n]], rows[b], gsem[b]).wait()

        def start_write(n, b):
            c, tc = unit_ct(n)
            for tr in range(4):
                pltpu.async_copy(
                    tbuf[b].at[pl.ds(tr * 1024, 1024)],
                    out_hbm.at[c, tr, tc], ssem[b])

        def wait_write(n, b):
            c, tc = unit_ct(n)
            for tr in range(4):
                pltpu.make_async_copy(
                    tbuf[b].at[pl.ds(tr * 1024, 1024)],
                    out_hbm.at[c, tr, tc], ssem[b]).wait()

        row_ids = [lax.iota(jnp.int32, 16) + 16 * h for h in range(8)]
        cols = [jnp.full((16,), j, jnp.int32) for j in range(D)]

        for b in range(NSLOT_B):
            start_gather(b, b)

        @pl.loop(0, U_PER_W, step=NSLOT_B)
        def _(n):
            for b in range(NSLOT_B):
                nb = n + b
                wait_gather(nb, b)
                @pl.when(nb >= NSLOT_B)
                def _():
                    wait_write(nb - NSLOT_B, b)
                for j2 in range(0, D, 2):
                    vals = [plsc.load_gather(rows[b], [row_ids[h], cols[j]])
                            for j in (j2, j2 + 1) for h in range(8)]
                    i = 0
                    for j in (j2, j2 + 1):
                        for h in range(8):
                            tbuf[b][pl.ds(j * 128 + 16 * h, 16)] = vals[i]
                            i += 1
                start_write(nb, b)
                @pl.when(nb + NSLOT_B < U_PER_W)
                def _():
                    start_gather(nb + NSLOT_B, b)

        for b in range(NSLOT_B):
            wait_write(U_PER_W - NSLOT_B + b, b)

    return k(table_r, idx_t)


def kernel(entity_indices, entity_embeddings):
    tail_flat = entity_embeddings[NFULL * 128 :].reshape(TAIL_E * D)
    table_r = _relayout(entity_embeddings.T, tail_flat).reshape(V, D)
    idx_t = entity_indices.T.reshape(NU, 128)
    out = _gather_t(table_r, idx_t)
    return (out.reshape(26, 4, 128, 8, 128)
            .transpose(2, 4, 0, 1, 3)
            .reshape(16384, 26, D))


# speedup vs baseline: 3.7140x; 3.7140x over previous
import functools

import jax
import jax.numpy as jnp
from jax import lax
from jax.experimental import pallas as pl
from jax.experimental.pallas import tpu as pltpu
from jax.experimental.pallas import tpu_sc as plsc

V = 1_000_000
D = 32
NC = 2
NS = 16
NW = NC * NS

CE = 128
NFULL = 7812
TAIL_E = V - NFULL * 128
NSLOT_A = 4

NU = 26 * 128
U_PER_W = NU // NW
NSLOT_B = 8


def _relayout(table_t, tail_flat):
    mesh = plsc.VectorSubcoreMesh(core_axis_name="c", subcore_axis_name="s")

    @functools.partial(
        pl.kernel,
        mesh=mesh,
        out_type=jax.ShapeDtypeStruct((V * D,), jnp.float32),
        compiler_params=pltpu.CompilerParams(needs_layout_passes=False),
        scratch_types=(
            [pltpu.VMEM((8, CE), jnp.float32) for _ in range(4 * NSLOT_A)]
            + [pltpu.VMEM((CE * D,), jnp.float32) for _ in range(NSLOT_A)]
            + [pltpu.SemaphoreType.DMA for _ in range(2 * NSLOT_A)]
        ),
    )
    def k(tab_hbm, tail_hbm, out_hbm, *scratch):
        strips = [scratch[4 * b : 4 * b + 4] for b in range(NSLOT_A)]
        rows = scratch[4 * NSLOT_A : 5 * NSLOT_A]
        gsem = scratch[5 * NSLOT_A : 6 * NSLOT_A]
        ssem = scratch[6 * NSLOT_A : 7 * NSLOT_A]

        wid = lax.axis_index("s") * NC + lax.axis_index("c")
        cstart = 244 * wid + 4 * jnp.minimum(wid, 1)
        nch = 244 + 4 * jnp.where(wid == 0, 1, 0)

        def start_reads(n, b):
            e0 = (cstart + n) * CE
            for tr in range(4):
                pltpu.async_copy(
                    tab_hbm.at[pl.ds(8 * tr, 8), pl.ds(e0, CE)],
                    strips[b][tr], gsem[b])

        def wait_reads(n, b):
            e0 = (cstart + n) * CE
            for tr in range(4):
                pltpu.make_async_copy(
                    tab_hbm.at[pl.ds(8 * tr, 8), pl.ds(e0, CE)],
                    strips[b][tr], gsem[b]).wait()

        def start_write(n, b):
            e0 = (cstart + n) * CE
            pltpu.async_copy(rows[b], out_hbm.at[pl.ds(e0 * D, CE * D)],
                             ssem[b])

        def wait_write(n, b):
            e0 = (cstart + n) * CE
            pltpu.make_async_copy(
                rows[b], out_hbm.at[pl.ds(e0 * D, CE * D)], ssem[b]).wait()

        ibase = lax.iota(jnp.int32, 16) * D

        iota16 = lax.iota(jnp.int32, 16)

        def transpose_chunk(b):
            @plsc.parallel_loop(0, 8, unroll=2)
            def _(l0):
                lvec = iota16 + l0 * 16
                lbase = lvec * D
                for dd in range(8):
                    rv = (iota16 + dd) & 7
                    for tr in range(4):
                        val = plsc.load_gather(strips[b][tr], [rv, lvec])
                        plsc.store_scatter(
                            rows[b], [lbase + (8 * tr) + rv], val)

        for b in range(NSLOT_A):
            start_reads(b, b)

        @pl.loop(0, nch, step=NSLOT_A)
        def _(ch):
            for b in range(NSLOT_A):
                n = ch + b
                wait_reads(n, b)
                @pl.when(n >= NSLOT_A)
                def _():
                    wait_write(n - NSLOT_A, b)
                transpose_chunk(b)
                start_write(n, b)
                @pl.when(n + NSLOT_A < nch)
                def _():
                    start_reads(n + NSLOT_A, b)

        for b in range(NSLOT_A):
            wait_write(nch - NSLOT_A + b, b)

        @pl.when(wid == 0)
        def _():
            pltpu.sync_copy(tail_hbm, rows[0].at[pl.ds(0, TAIL_E * D)])
            pltpu.sync_copy(rows[0].at[pl.ds(0, TAIL_E * D)],
                            out_hbm.at[pl.ds(NFULL * 128 * D, TAIL_E * D)])

    return k(table_t, tail_flat)


def _gather_t(table_r, idx_t):
    mesh = plsc.VectorSubcoreMesh(core_axis_name="c", subcore_axis_name="s")

    @functools.partial(
        pl.kernel,
        mesh=mesh,
        out_type=jax.ShapeDtypeStruct((26, 4, 128, 1024), jnp.float32),
        compiler_params=pltpu.CompilerParams(
            use_tc_tiling_on_sc=False, needs_layout_passes=False),
        scratch_types=(
            [pltpu.VMEM((U_PER_W, 128), jnp.int32)]
            + [pltpu.VMEM((128, D), jnp.float32) for _ in range(NSLOT_B)]
            + [pltpu.VMEM((4096,), jnp.float32) for _ in range(NSLOT_B)]
            + [pltpu.SemaphoreType.DMA for _ in range(2 * NSLOT_B)]
        ),
    )
    def k(table_hbm, idx_hbm, out_hbm, *scratch):
        idx_v = scratch[0]
        rows = scratch[1 : 1 + NSLOT_B]
        tbuf = scratch[1 + NSLOT_B : 1 + 2 * NSLOT_B]
        gsem = scratch[1 + 2 * NSLOT_B : 1 + 3 * NSLOT_B]
        ssem = scratch[1 + 3 * NSLOT_B : 1 + 4 * NSLOT_B]

        wid = lax.axis_index("s") * NC + lax.axis_index("c")
        ubase = wid * U_PER_W
        pltpu.sync_copy(idx_hbm.at[pl.ds(ubase, U_PER_W)], idx_v)

        def unit_ct(n):
            u = ubase + n
            return lax.div(u, 128), lax.rem(u, 128)

        def start_gather(n, b):
            pltpu.async_copy(table_hbm.at[idx_v.at[n]], rows[b], gsem[b])

        def wait_gather(n, b):
            pltpu.make_async_copy(
                table_hbm.at[idx_v.at[n]], rows[b], gsem[b]).wait()

        def start_write(n, b):
            c, tc = unit_ct(n)
            for tr in range(4):
                pltpu.async_copy(
                    tbuf[b].at[pl.ds(tr * 1024, 1024)],
                    out_hbm.at[c, tr, tc], ssem[b])

        def wait_write(n, b):
            c, tc = unit_ct(n)
            for tr in range(4):
                pltpu.make_async_copy(
                    tbuf[b].at[pl.ds(tr * 1024, 1024)],
                    out_hbm.at[c, tr, tc], ssem[b]).wait()

        row_ids = [lax.iota(jnp.int32, 16) + 16 * h for h in range(8)]
        cols = [jnp.full((16,), j, jnp.int32) for j in range(D)]

        for b in range(NSLOT_B):
            start_gather(b, b)

        @pl.loop(0, U_PER_W, step=NSLOT_B)
        def _(n):
            for b in range(NSLOT_B):
                nb = n + b
                wait_gather(nb, b)
                @pl.when(nb >= NSLOT_B)
                def _():
                    wait_write(nb - NSLOT_B, b)
                @plsc.parallel_loop(0, D, unroll=2)
                def _(j):
                    vals, sidx = [], []
                    for h in range(8):
                        lv = row_ids[h]
                        cv = (lv + j) & 31
                        vals.append(plsc.load_gather(rows[b], [lv, cv]))
                        sidx.append(cv * 128 + lv)
                    for h in range(8):
                        plsc.store_scatter(tbuf[b], [sidx[h]], vals[h])
                start_write(nb, b)
                @pl.when(nb + NSLOT_B < U_PER_W)
                def _():
                    start_gather(nb + NSLOT_B, b)

        for b in range(NSLOT_B):
            wait_write(U_PER_W - NSLOT_B + b, b)

    return k(table_r, idx_t)


def kernel(entity_indices, entity_embeddings):
    tail_flat = entity_embeddings[NFULL * 128 :].reshape(TAIL_E * D)
    table_r = _relayout(entity_embeddings.T, tail_flat).reshape(V, D)
    idx_t = entity_indices.T.reshape(NU, 128)
    out = _gather_t(table_r, idx_t)
    return (out.reshape(26, 4, 128, 8, 128)
            .transpose(2, 4, 0, 1, 3)
            .reshape(16384, 26, D))


# speedup vs baseline: 3.7354x; 1.0058x over previous
import functools

import jax
import jax.numpy as jnp
from jax import lax
from jax.experimental import pallas as pl
from jax.experimental.pallas import tpu as pltpu
from jax.experimental.pallas import tpu_sc as plsc

V = 1_000_000
D = 32
NC = 2
NS = 16
NW = NC * NS

CE = 128
NFULL = 7812
TAIL_E = V - NFULL * 128
NSLOT_A = 4

NU = 26 * 128
U_PER_W = NU // NW
NSLOT_B = 8


def _relayout(table_t, tail_flat):
    mesh = plsc.VectorSubcoreMesh(core_axis_name="c", subcore_axis_name="s")

    @functools.partial(
        pl.kernel,
        mesh=mesh,
        out_type=jax.ShapeDtypeStruct((V * D,), jnp.float32),
        compiler_params=pltpu.CompilerParams(needs_layout_passes=False),
        scratch_types=(
            [pltpu.VMEM((8, CE), jnp.float32) for _ in range(4 * NSLOT_A)]
            + [pltpu.VMEM((CE * D,), jnp.float32) for _ in range(NSLOT_A)]
            + [pltpu.SemaphoreType.DMA for _ in range(2 * NSLOT_A)]
        ),
    )
    def k(tab_hbm, tail_hbm, out_hbm, *scratch):
        strips = [scratch[4 * b : 4 * b + 4] for b in range(NSLOT_A)]
        rows = scratch[4 * NSLOT_A : 5 * NSLOT_A]
        gsem = scratch[5 * NSLOT_A : 6 * NSLOT_A]
        ssem = scratch[6 * NSLOT_A : 7 * NSLOT_A]

        wid = lax.axis_index("s") * NC + lax.axis_index("c")
        cstart = 244 * wid + 4 * jnp.minimum(wid, 1)
        nch = 244 + 4 * jnp.where(wid == 0, 1, 0)

        def start_reads(n, b):
            e0 = (cstart + n) * CE
            for tr in range(4):
                pltpu.async_copy(
                    tab_hbm.at[pl.ds(8 * tr, 8), pl.ds(e0, CE)],
                    strips[b][tr], gsem[b])

        def wait_reads(n, b):
            e0 = (cstart + n) * CE
            for tr in range(4):
                pltpu.make_async_copy(
                    tab_hbm.at[pl.ds(8 * tr, 8), pl.ds(e0, CE)],
                    strips[b][tr], gsem[b]).wait()

        def start_write(n, b):
            e0 = (cstart + n) * CE
            pltpu.async_copy(rows[b], out_hbm.at[pl.ds(e0 * D, CE * D)],
                             ssem[b])

        def wait_write(n, b):
            e0 = (cstart + n) * CE
            pltpu.make_async_copy(
                rows[b], out_hbm.at[pl.ds(e0 * D, CE * D)], ssem[b]).wait()

        iota16 = lax.iota(jnp.int32, 16)

        def transpose_chunk(b):
            @plsc.parallel_loop(0, 8, unroll=2)
            def _(l0):
                lvec = iota16 + l0 * 16
                lbase = lvec * D
                for dd in range(8):
                    rv = (iota16 + dd) & 7
                    for tr in range(4):
                        val = plsc.load_gather(strips[b][tr], [rv, lvec])
                        plsc.store_scatter(
                            rows[b], [lbase + (8 * tr) + rv], val)

        for b in range(NSLOT_A):
            start_reads(b, b)

        @pl.loop(0, nch, step=NSLOT_A)
        def _(ch):
            for b in range(NSLOT_A):
                n = ch + b
                wait_reads(n, b)
                @pl.when(n >= NSLOT_A)
                def _():
                    wait_write(n - NSLOT_A, b)
                transpose_chunk(b)
                start_write(n, b)
                @pl.when(n + NSLOT_A < nch)
                def _():
                    start_reads(n + NSLOT_A, b)

        for b in range(NSLOT_A):
            wait_write(nch - NSLOT_A + b, b)

        @pl.when(wid == 0)
        def _():
            pltpu.sync_copy(tail_hbm, rows[0].at[pl.ds(0, TAIL_E * D)])
            pltpu.sync_copy(rows[0].at[pl.ds(0, TAIL_E * D)],
                            out_hbm.at[pl.ds(NFULL * 128 * D, TAIL_E * D)])

    return k(table_t, tail_flat)


def _gather_t(table_r, idx_t):
    mesh = plsc.VectorSubcoreMesh(core_axis_name="c", subcore_axis_name="s")

    @functools.partial(
        pl.kernel,
        mesh=mesh,
        out_type=jax.ShapeDtypeStruct((26, 4, 128, 1024), jnp.float32),
        compiler_params=pltpu.CompilerParams(
            use_tc_tiling_on_sc=False, needs_layout_passes=False),
        scratch_types=(
            [pltpu.VMEM((U_PER_W, 128), jnp.int32)]
            + [pltpu.VMEM((128, D), jnp.float32) for _ in range(NSLOT_B)]
            + [pltpu.VMEM((4096,), jnp.float32) for _ in range(NSLOT_B)]
            + [pltpu.SemaphoreType.DMA for _ in range(2 * NSLOT_B)]
        ),
    )
    def k(table_hbm, idx_hbm, out_hbm, *scratch):
        idx_v = scratch[0]
        rows = scratch[1 : 1 + NSLOT_B]
        tbuf = scratch[1 + NSLOT_B : 1 + 2 * NSLOT_B]
        gsem = scratch[1 + 2 * NSLOT_B : 1 + 3 * NSLOT_B]
        ssem = scratch[1 + 3 * NSLOT_B : 1 + 4 * NSLOT_B]

        wid = lax.axis_index("s") * NC + lax.axis_index("c")
        ubase = wid * U_PER_W
        pltpu.sync_copy(idx_hbm.at[pl.ds(ubase, U_PER_W)], idx_v)

        def unit_ct(n):
            u = ubase + n
            return lax.div(u, 128), lax.rem(u, 128)

        def start_gather(n, b):
            pltpu.async_copy(table_hbm.at[idx_v.at[n]], rows[b], gsem[b])

        def wait_gather(n, b):
            pltpu.make_async_copy(
                table_hbm.at[idx_v.at[n]], rows[b], gsem[b]).wait()

        def start_write(n, b):
            c, tc = unit_ct(n)
            for tr in range(4):
                pltpu.async_copy(
                    tbuf[b].at[pl.ds(tr * 1024, 1024)],
                    out_hbm.at[c, tr, tc], ssem[b])

        def wait_write(n, b):
            c, tc = unit_ct(n)
            for tr in range(4):
                pltpu.make_async_copy(
                    tbuf[b].at[pl.ds(tr * 1024, 1024)],
                    out_hbm.at[c, tr, tc], ssem[b]).wait()

        row_ids = [lax.iota(jnp.int32, 16) + 16 * h for h in range(8)]

        for b in range(NSLOT_B):
            start_gather(b, b)

        @pl.loop(0, U_PER_W, step=NSLOT_B)
        def _(n):
            for b in range(NSLOT_B):
                nb = n + b
                wait_gather(nb, b)
                @pl.when(nb >= NSLOT_B)
                def _():
                    wait_write(nb - NSLOT_B, b)
                @plsc.parallel_loop(0, D, unroll=2)
                def _(j):
                    vals, sidx = [], []
                    for h in range(8):
                        lv = row_ids[h]
                        cv = (lv + j) & 31
                        vals.append(plsc.load_gather(rows[b], [lv, cv]))
                        sidx.append(cv * 128 + lv)
                    for h in range(8):
                        plsc.store_scatter(tbuf[b], [sidx[h]], vals[h])
                start_write(nb, b)
                @pl.when(nb + NSLOT_B < U_PER_W)
                def _():
                    start_gather(nb + NSLOT_B, b)

        for b in range(NSLOT_B):
            wait_write(U_PER_W - NSLOT_B + b, b)

    return k(table_r, idx_t)


def kernel(entity_indices, entity_embeddings):
    tail_flat = entity_embeddings[NFULL * 128 :].reshape(TAIL_E * D)
    table_r = _relayout(entity_embeddings.T, tail_flat).reshape(V, D)
    idx_t = entity_indices.T.reshape(NU, 128)
    out = _gather_t(table_r, idx_t)
    return (out.reshape(26, 4, 128, 8, 128)
            .transpose(2, 4, 0, 1, 3)
            .reshape(16384, 26, D))
